# Initial kernel scaffold; baseline (speedup 1.0000x reference)
import jax
import jax.numpy as jnp
from jax.experimental import pallas as pl


def kernel(x, edge_index, batch, W1, b1, W2, b2, Wm1, bm1, Wm2, bm2, prototypes):
    raise NotImplementedError("write your pallas kernel here")


# speedup vs baseline: 8.5122x; 8.5122x over previous
import functools

import jax
import jax.numpy as jnp
from jax import lax
from jax.experimental import pallas as pl
from jax.experimental.pallas import tpu as pltpu
from jax.experimental.pallas import tpu_sc as plsc

_N = 10000
_E = 320000
_D = 128
_G = 128
_P = 16
_EPS = 1e-07
_R = 0.5

_NC = 2
_NS = 16
_NW = _NC * _NS
_EPT = _E // _NW
_ESS = _E // _NS
_K = 80
_NCHUNK = _ESS // _K
_NPASS = 1
_NR = 5120
_NP = _NC * _NPASS * _NR
_ACC = _NR + 8
_RPT = _NR // _NS
_BC = 10
_EBK = 2000
_EBC = _EPT // _EBK
_NBLK = _NCHUNK // _BC


def _segsum_body(table_hbm, src_hbm, dst_hbm, out_hbm,
                 src_v, dstb, rows_v, zbuf, acc, semi0, semi1, semg0, semg1):
    cid = lax.axis_index("c")
    sid = lax.axis_index("s")
    semis = (semi0, semi1)
    semgs = (semg0, semg1)
    row0 = sid * _RPT

    pltpu.sync_copy(src_hbm.at[sid], src_v)

    def _zrow(i, c):
        for j in range(_D // 16):
            zbuf[i, pl.ds(j * 16, 16)] = jnp.zeros((16,), jnp.float32)
        return c
    lax.fori_loop(0, 8, _zrow, None)

    def _gather_start(ci, b):
        pltpu.make_async_copy(
            table_hbm.at[src_v.at[ci]], rows_v.at[b], semgs[b]).start()

    def _gather_wait(ci, b):
        pltpu.make_async_copy(
            table_hbm.at[src_v.at[ci]], rows_v.at[b], semgs[b]).wait()

    def _idx_start(blk, s):
        pltpu.make_async_copy(dst_hbm.at[sid, blk], dstb.at[s], semis[s]).start()

    def _idx_wait(blk, s):
        pltpu.make_async_copy(dst_hbm.at[sid, blk], dstb.at[s], semis[s]).wait()

    for p in range(_NPASS):
        lo = (cid * _NPASS + p) * _NR

        for q in range(_RPT // 8):
            pltpu.sync_copy(zbuf, acc.at[pl.ds(row0 + q * 8, 8)])

        @pl.when(sid == 0)
        def _():
            pltpu.sync_copy(zbuf, acc.at[pl.ds(_NR, 8)])

        plsc.subcore_barrier()

        _idx_start(0, 0)
        _idx_start(1, 1)
        _gather_start(0, 0)
        _gather_start(1, 1)

        def _block(blk, s):
            _idx_wait(blk, s)

            def _remap(i, c):
                r = i // (_K // 16)
                col = (i % (_K // 16)) * 16
                d = dstb[s, r, pl.ds(col, 16)] - lo
                ok = (d >= 0) & (d < _NR)
                dstb[s, r, pl.ds(col, 16)] = jnp.where(ok, d, _NR)
                return c
            lax.fori_loop(0, _BC * (_K // 16), _remap, None)

            def _chunkpair(u, c):
                for b in range(2):
                    j = 2 * u + b
                    ci = blk * _BC + j
                    _gather_wait(ci, b)
                    pltpu.sync_copy(rows_v.at[b], acc.at[dstb.at[s, j]],
                                    add=True)
                    nxt = ci + 2

                    @pl.when(nxt < _NCHUNK)
                    def _():
                        _gather_start(nxt, b)
                return c
            lax.fori_loop(0, _BC // 2, _chunkpair, None)

        def _blockpair(t, c):
            for s in range(2):
                blk = 2 * t + s
                _block(blk, s)
                nxtb = blk + 2

                @pl.when(nxtb < _NBLK)
                def _():
                    _idx_start(nxtb, s)
            return c
        lax.fori_loop(0, _NBLK // 2, _blockpair, None)
        _block(_NBLK - 1, 0)

        plsc.subcore_barrier()
        pltpu.sync_copy(acc.at[pl.ds(row0, _RPT)],
                        out_hbm.at[pl.ds(lo + row0, _RPT)])
        plsc.subcore_barrier()


def _sc_segsum(table, src3d, dst4d):
    kern = pl.kernel(
        _segsum_body,
        out_type=jax.ShapeDtypeStruct((_NP, _D), jnp.float32),
        mesh=plsc.VectorSubcoreMesh(core_axis_name="c", subcore_axis_name="s"),
        scratch_types=[
            pltpu.VMEM((_NCHUNK, _K), jnp.int32),
            pltpu.VMEM((2, _BC, _K), jnp.int32),
            pltpu.VMEM((2, _K, _D), jnp.float32),
            pltpu.VMEM((8, _D), jnp.float32),
            pltpu.VMEM_SHARED((_ACC, _D), jnp.float32),
            pltpu.SemaphoreType.DMA,
            pltpu.SemaphoreType.DMA,
            pltpu.SemaphoreType.DMA,
            pltpu.SemaphoreType.DMA,
        ],
    )
    return kern(table, src3d, dst4d)


def _edge_bern_body(nb_hbm, src_hbm, dst_hbm, out_hbm,
                    si0, si1, di0, di1, a0, a1, b0, b1, o0, o1,
                    semi0, semi1, sg0, sg1):
    cid = lax.axis_index("c")
    sid = lax.axis_index("s")
    wid = sid * _NC + cid
    si = (si0, si1)
    di = (di0, di1)
    av = (a0, a1)
    bv = (b0, b1)
    ov = (o0, o1)
    semi = (semi0, semi1)
    sg = (sg0, sg1)
    nch = _EPT // _K

    def _fire_idx(ci, s):
        pltpu.make_async_copy(src_hbm.at[wid, ci], si[s], semi[s]).start()
        pltpu.make_async_copy(dst_hbm.at[wid, ci], di[s], semi[s]).start()

    def _drain_idx(ci, s):
        pltpu.make_async_copy(src_hbm.at[wid, ci], si[s], semi[s]).wait()
        pltpu.make_async_copy(dst_hbm.at[wid, ci], di[s], semi[s]).wait()

    def _fire_g(s):
        pltpu.make_async_copy(nb_hbm.at[si[s].at[0]], av[s], sg[s]).start()
        pltpu.make_async_copy(nb_hbm.at[di[s].at[0]], bv[s], sg[s]).start()

    def _drain_g(s):
        pltpu.make_async_copy(nb_hbm.at[si[s].at[0]], av[s], sg[s]).wait()
        pltpu.make_async_copy(nb_hbm.at[di[s].at[0]], bv[s], sg[s]).wait()

    _fire_idx(0, 0)
    _drain_idx(0, 0)
    _fire_g(0)
    _fire_idx(1, 1)
    for ci in range(nch):
        s = ci % 2
        if ci + 1 < nch:
            _drain_idx(ci + 1, 1 - s)
            _fire_g(1 - s)
        _drain_g(s)
        if ci + 2 < nch:
            _fire_idx(ci + 2, s)

        def _edge(r, c):
            ov[s][r] = av[s][r, pl.ds(0, 16)] * bv[s][r, pl.ds(0, 16)]
            return c
        lax.fori_loop(0, _K, _edge, None)
        pltpu.sync_copy(ov[s], out_hbm.at[wid, ci])


def _sc_edge_bern(nbrep, src4, dst4):
    kern = pl.kernel(
        _edge_bern_body,
        out_type=jax.ShapeDtypeStruct((_NW, _EPT // _K, _K, 16), jnp.float32),
        mesh=plsc.VectorSubcoreMesh(core_axis_name="c", subcore_axis_name="s"),
        scratch_types=[
            pltpu.VMEM((1, _K), jnp.int32),
            pltpu.VMEM((1, _K), jnp.int32),
            pltpu.VMEM((1, _K), jnp.int32),
            pltpu.VMEM((1, _K), jnp.int32),
            pltpu.VMEM((_K, _D), jnp.float32),
            pltpu.VMEM((_K, _D), jnp.float32),
            pltpu.VMEM((_K, _D), jnp.float32),
            pltpu.VMEM((_K, _D), jnp.float32),
            pltpu.VMEM((_K, 16), jnp.float32),
            pltpu.VMEM((_K, 16), jnp.float32),
            pltpu.SemaphoreType.DMA,
            pltpu.SemaphoreType.DMA,
            pltpu.SemaphoreType.DMA,
            pltpu.SemaphoreType.DMA,
        ],
    )
    return kern(nbrep, src4, dst4)


def _dot(a, b):
    return jax.lax.dot_general(a, b, (((1,), (0,)), ((), ())),
                               preferred_element_type=jnp.float32)


def _dot_t(a, b):
    return jax.lax.dot_general(a, b, (((1,), (1,)), ((), ())),
                               preferred_element_type=jnp.float32)


def _dot_tn(a, b):
    return jax.lax.dot_general(a, b, (((0,), (0,)), ((), ())),
                               preferred_element_type=jnp.float32)


def _rownorm(a):
    n = jnp.sqrt(jnp.sum(a * a, axis=1, keepdims=True))
    return jnp.where(n == 0.0, n + _EPS, n)


def _tc_layer_body(x_ref, s_ref, w_ref, b_ref, o_ref):
    t = x_ref[...] + s_ref[:_N]
    o_ref[...] = jnp.maximum(_dot(t, w_ref[...]) + b_ref[...], 0.0)


def _tc_layer(x, s, W, b2d):
    return pl.pallas_call(
        _tc_layer_body,
        out_shape=jax.ShapeDtypeStruct((_N, _D), jnp.float32),
    )(x, s, W, b2d)


def _tc_stage2_body(h1_ref, s_ref, x_ref, batch_ref, prot_ref, w2_ref, b2_ref,
                    wm1_ref, bm1_ref, wm2_ref, bm2_ref,
                    h2_ref, nb_ref, y_ref):
    h2 = jnp.maximum(_dot(h1_ref[...] + s_ref[:_N], w2_ref[...])
                     + b2_ref[...], 0.0)
    h2_ref[...] = h2

    gi = lax.broadcasted_iota(jnp.int32, (1, _G), 1)
    oh = (batch_ref[...] == gi).astype(jnp.float32)
    ge = _dot_tn(oh, h2)

    prot = prot_ref[...]
    gn = _rownorm(ge)
    pn = _rownorm(prot)
    sim0 = _dot_t(ge, prot) / _dot_t(gn, pn)

    mx = jnp.max(sim0, axis=1, keepdims=True)
    pi = lax.broadcasted_iota(jnp.int32, (_G, _P), 1)
    assign = jnp.min(jnp.where(sim0 >= mx, pi, _P), axis=1, keepdims=True)
    oh_a = (assign == pi).astype(jnp.float32)
    p_assigned = _dot(oh_a, prot)

    wm1 = wm1_ref[...]
    pergraph = _dot(p_assigned, wm1[_D:, :])
    t = jnp.maximum(_dot(h2, wm1[:_D, :]) + _dot(oh, pergraph) + bm1_ref[...],
                    0.0)
    prob = _dot(t, wm2_ref[...]) + bm2_ref[...]
    nb = jax.nn.sigmoid(prob)
    nb_ref[...] = nb
    y_ref[...] = x_ref[...] * (nb * nb)


def _tc_stage2(h1, s2, x, batch2d, prot, W2, b2d, Wm1, bm1d, Wm2, bm2d):
    return pl.pallas_call(
        _tc_stage2_body,
        out_shape=[
            jax.ShapeDtypeStruct((_N, _D), jnp.float32),
            jax.ShapeDtypeStruct((_N, 1), jnp.float32),
            jax.ShapeDtypeStruct((_N, _D), jnp.float32),
        ],
    )(h1, s2, x, batch2d, prot, W2, b2d, Wm1, bm1d, Wm2, bm2d)


def _tc_stage3_body(x_ref, s_ref, nb_ref, w1_ref, b1_ref, g1_ref, z_ref):
    nb = nb_ref[...]
    t = nb * (x_ref[...] + s_ref[:_N])
    g1 = jnp.maximum(_dot(t, w1_ref[...]) + b1_ref[...], 0.0)
    g1_ref[...] = g1
    z_ref[...] = g1 * nb


def _tc_stage3(x, s3, nb, W1, b1d):
    return pl.pallas_call(
        _tc_stage3_body,
        out_shape=[
            jax.ShapeDtypeStruct((_N, _D), jnp.float32),
            jax.ShapeDtypeStruct((_N, _D), jnp.float32),
        ],
    )(x, s3, nb, W1, b1d)


def _tc_stage4_body(g1_ref, s_ref, nb_ref, eb_ref, batch_ref, prot_ref,
                    w2_ref, b2_ref,
                    se_ref, sim_ref, dsim_ref, kl_ref, nce_ref):
    nb = nb_ref[...]
    t = g1_ref[...] + nb * s_ref[:_N]
    g2 = jnp.maximum(_dot(t, w2_ref[...]) + b2_ref[...], 0.0)

    gi = lax.broadcasted_iota(jnp.int32, (1, _G), 1)
    oh = (batch_ref[...] == gi).astype(jnp.float32)
    se = _dot_tn(oh, g2)
    se_ref[...] = se

    prot = prot_ref[...]
    sn = _rownorm(se)
    pn = _rownorm(prot)
    sim = _dot_t(se, prot) / _dot_t(sn, pn)
    sim_ref[...] = sim
    dsim_ref[...] = _dot_t(se, se) / _dot_t(sn, sn)

    mx = jnp.max(sim, axis=1, keepdims=True)
    pi = lax.broadcasted_iota(jnp.int32, (_G, _P), 1)
    assign = jnp.min(jnp.where(sim >= mx, pi, _P), axis=1, keepdims=True)
    oh_a = (assign == pi).astype(jnp.float32)
    s = jnp.exp(sim * 5.0)
    pos = jnp.sum(s * oh_a, axis=1, keepdims=True)
    neg = jnp.sum(s * (1.0 - oh_a), axis=1, keepdims=True)
    nce = -jnp.mean(jnp.log(pos / neg))
    nce_ref[...] = nce.reshape(1, 1)

    kn = jnp.mean(nb * jnp.log(nb / _R + _EPS)
                  + (1.0 - nb) * jnp.log((1.0 - nb) / (1.0 - _R + _EPS) + _EPS))
    eb = eb_ref[...]
    rr = _R * _R
    ke = jnp.mean(eb * jnp.log(eb / rr + _EPS)
                  + (1.0 - eb) * jnp.log((1.0 - eb) / (1.0 - rr + _EPS) + _EPS))
    kl_ref[...] = (kn + ke).reshape(1, 1)


def _tc_stage4(g1, s4, nb, eb2d, batch2d, prot, W2, b2d):
    return pl.pallas_call(
        _tc_stage4_body,
        out_shape=[
            jax.ShapeDtypeStruct((_G, _D), jnp.float32),
            jax.ShapeDtypeStruct((_G, _P), jnp.float32),
            jax.ShapeDtypeStruct((_G, _G), jnp.float32),
            jax.ShapeDtypeStruct((1, 1), jnp.float32),
            jax.ShapeDtypeStruct((1, 1), jnp.float32),
        ],
    )(g1, s4, nb, eb2d, batch2d, prot, W2, b2d)


def kernel(x, edge_index, batch, W1, b1, W2, b2, Wm1, bm1, Wm2, bm2,
           prototypes):
    src = edge_index[0]
    dst = edge_index[1]
    src3d = src.reshape(_NS, _NCHUNK, _K)
    dst4d = dst.reshape(_NS, _NBLK, _BC, _K)
    batch2d = batch.reshape(_N, 1)
    b1d = b1.reshape(1, _D)
    b2d = b2.reshape(1, _D)
    bm1d = bm1.reshape(1, _D)
    bm2d = bm2.reshape(1, 1)

    s1 = _sc_segsum(x, src3d, dst4d)
    h1 = _tc_layer(x, s1, W1, b1d)
    s2 = _sc_segsum(h1, src3d, dst4d)
    h2, nb, y = _tc_stage2(h1, s2, x, batch2d, prototypes, W2, b2d,
                           Wm1, bm1d, Wm2, bm2d)
    nbrep = jnp.broadcast_to(nb, (_N, _D))
    eb = _sc_edge_bern(nbrep, src.reshape(_NW, _EPT // _K, 1, _K),
                       dst.reshape(_NW, _EPT // _K, 1, _K))
    eb = eb.reshape(_E, 16)[:, 0]
    s3 = _sc_segsum(y, src3d, dst4d)
    g1, z = _tc_stage3(x, s3, nb, W1, b1d)
    s4 = _sc_segsum(z, src3d, dst4d)
    se, sim, dsim, kl, nce = _tc_stage4(g1, s4, nb, eb.reshape(_E // _D, _D),
                                        batch2d, prototypes, W2, b2d)

    return (kl[0, 0], nce[0, 0], sim, nb, eb.reshape(_E, 1), dsim, se, h2)


# speedup vs baseline: 13.1877x; 1.5493x over previous
import functools

import jax
import jax.numpy as jnp
from jax import lax
from jax.experimental import pallas as pl
from jax.experimental.pallas import tpu as pltpu
from jax.experimental.pallas import tpu_sc as plsc

_N = 10000
_E = 320000
_D = 128
_G = 128
_P = 16
_EPS = 1e-07
_R = 0.5

_NC = 2
_NS = 16
_NW = _NC * _NS
_EPT = _E // _NW
_ESS = _E // _NS
_K = 80
_NCHUNK = _ESS // _K
_EPW = _E // _NW
_NCH2 = _EPW // _K
_BC2 = 5
_NB2 = _NCH2 // _BC2
_NP = 10240
_RPT = _NP // _NS


def _segsum_body(table_hbm, src_hbm, dst_hbm, out_hbm,
                 src_v, dstb, rows_v, zbuf, acc, semi0, semi1, semg0, semg1):
    cid = lax.axis_index("c")
    sid = lax.axis_index("s")
    wid = sid * _NC + cid
    semis = (semi0, semi1)
    semgs = (semg0, semg1)
    row0 = sid * _RPT

    pltpu.sync_copy(src_hbm.at[wid], src_v)

    def _zrow(i, c):
        for j in range(_D // 16):
            zbuf[i, pl.ds(j * 16, 16)] = jnp.zeros((16,), jnp.float32)
        return c
    lax.fori_loop(0, 8, _zrow, None)
    for q in range(_RPT // 8):
        pltpu.sync_copy(zbuf, acc.at[pl.ds(row0 + q * 8, 8)])

    plsc.subcore_barrier()

    def _g_start(ci, b):
        pltpu.make_async_copy(
            table_hbm.at[src_v.at[ci]], rows_v.at[b], semgs[b]).start()

    def _g_wait(ci, b):
        pltpu.make_async_copy(
            table_hbm.at[src_v.at[ci]], rows_v.at[b], semgs[b]).wait()

    def _idx_start(blk, s):
        pltpu.make_async_copy(
            dst_hbm.at[wid, blk], dstb.at[s], semis[s]).start()

    def _idx_wait(blk, s):
        pltpu.make_async_copy(
            dst_hbm.at[wid, blk], dstb.at[s], semis[s]).wait()

    _idx_start(0, 0)
    _idx_start(1, 1)
    _g_start(0, 0)
    _g_start(1, 1)

    def _block(blk, s, par):
        _idx_wait(blk, s)
        for j in range(_BC2):
            ci = blk * _BC2 + j
            b = (par + j) % 2
            _g_wait(ci, b)
            pltpu.sync_copy(rows_v.at[b], acc.at[dstb.at[s, j]], add=True)
            nxt = ci + 2

            @pl.when(nxt < _NCH2)
            def _():
                _g_start(nxt, b)

    def _blockpair(t, c):
        for sblk in range(2):
            blk = 2 * t + sblk
            _block(blk, sblk, sblk)
            nxtb = blk + 2

            @pl.when(nxtb < _NB2)
            def _():
                _idx_start(nxtb, sblk)
        return c
    lax.fori_loop(0, _NB2 // 2, _blockpair, None)
    _block(_NB2 - 1, 0, 0)

    plsc.subcore_barrier()
    pltpu.sync_copy(acc.at[pl.ds(row0, _RPT)],
                    out_hbm.at[cid, pl.ds(row0, _RPT)])


def _sc_segsum(table, src3d, dst4d):
    kern = pl.kernel(
        _segsum_body,
        out_type=jax.ShapeDtypeStruct((_NC, _NP, _D), jnp.float32),
        mesh=plsc.VectorSubcoreMesh(core_axis_name="c", subcore_axis_name="s"),
        scratch_types=[
            pltpu.VMEM((_NCH2, _K), jnp.int32),
            pltpu.VMEM((2, _BC2, _K), jnp.int32),
            pltpu.VMEM((2, _K, _D), jnp.float32),
            pltpu.VMEM((8, _D), jnp.float32),
            pltpu.VMEM_SHARED((_NP, _D), jnp.float32),
            pltpu.SemaphoreType.DMA,
            pltpu.SemaphoreType.DMA,
            pltpu.SemaphoreType.DMA,
            pltpu.SemaphoreType.DMA,
        ],
    )
    return kern(table, src3d, dst4d)


def _edge_bern_body(nb_hbm, src_hbm, dst_hbm, out_hbm,
                    si0, si1, di0, di1, a0, a1, b0, b1, o0, o1,
                    semi0, semi1, sg0, sg1):
    cid = lax.axis_index("c")
    sid = lax.axis_index("s")
    wid = sid * _NC + cid
    si = (si0, si1)
    di = (di0, di1)
    av = (a0, a1)
    bv = (b0, b1)
    ov = (o0, o1)
    semi = (semi0, semi1)
    sg = (sg0, sg1)
    nch = _EPT // _K

    def _fire_idx(ci, s):
        pltpu.make_async_copy(src_hbm.at[wid, ci], si[s], semi[s]).start()
        pltpu.make_async_copy(dst_hbm.at[wid, ci], di[s], semi[s]).start()

    def _drain_idx(ci, s):
        pltpu.make_async_copy(src_hbm.at[wid, ci], si[s], semi[s]).wait()
        pltpu.make_async_copy(dst_hbm.at[wid, ci], di[s], semi[s]).wait()

    def _fire_g(s):
        pltpu.make_async_copy(nb_hbm.at[si[s].at[0]], av[s], sg[s]).start()
        pltpu.make_async_copy(nb_hbm.at[di[s].at[0]], bv[s], sg[s]).start()

    def _drain_g(s):
        pltpu.make_async_copy(nb_hbm.at[si[s].at[0]], av[s], sg[s]).wait()
        pltpu.make_async_copy(nb_hbm.at[di[s].at[0]], bv[s], sg[s]).wait()

    _fire_idx(0, 0)
    _drain_idx(0, 0)
    _fire_g(0)
    _fire_idx(1, 1)
    for ci in range(nch):
        s = ci % 2
        if ci + 1 < nch:
            _drain_idx(ci + 1, 1 - s)
            _fire_g(1 - s)
        _drain_g(s)
        if ci + 2 < nch:
            _fire_idx(ci + 2, s)

        def _edge(r, c):
            ov[s][r] = av[s][r, pl.ds(0, 16)] * bv[s][r, pl.ds(0, 16)]
            return c
        lax.fori_loop(0, _K, _edge, None)
        pltpu.sync_copy(ov[s], out_hbm.at[wid, ci])


def _sc_edge_bern(nbrep, src4, dst4):
    kern = pl.kernel(
        _edge_bern_body,
        out_type=jax.ShapeDtypeStruct((_NW, _EPT // _K, _K, 16), jnp.float32),
        mesh=plsc.VectorSubcoreMesh(core_axis_name="c", subcore_axis_name="s"),
        scratch_types=[
            pltpu.VMEM((1, _K), jnp.int32),
            pltpu.VMEM((1, _K), jnp.int32),
            pltpu.VMEM((1, _K), jnp.int32),
            pltpu.VMEM((1, _K), jnp.int32),
            pltpu.VMEM((_K, _D), jnp.float32),
            pltpu.VMEM((_K, _D), jnp.float32),
            pltpu.VMEM((_K, _D), jnp.float32),
            pltpu.VMEM((_K, _D), jnp.float32),
            pltpu.VMEM((_K, 16), jnp.float32),
            pltpu.VMEM((_K, 16), jnp.float32),
            pltpu.SemaphoreType.DMA,
            pltpu.SemaphoreType.DMA,
            pltpu.SemaphoreType.DMA,
            pltpu.SemaphoreType.DMA,
        ],
    )
    return kern(nbrep, src4, dst4)


def _dot(a, b):
    return jax.lax.dot_general(a, b, (((1,), (0,)), ((), ())),
                               preferred_element_type=jnp.float32)


def _dot_t(a, b):
    return jax.lax.dot_general(a, b, (((1,), (1,)), ((), ())),
                               preferred_element_type=jnp.float32)


def _dot_tn(a, b):
    return jax.lax.dot_general(a, b, (((0,), (0,)), ((), ())),
                               preferred_element_type=jnp.float32)


def _rownorm(a):
    n = jnp.sqrt(jnp.sum(a * a, axis=1, keepdims=True))
    return jnp.where(n == 0.0, n + _EPS, n)


def _tc_layer_body(x_ref, s_ref, w_ref, b_ref, o_ref):
    t = x_ref[...] + s_ref[0, :_N] + s_ref[1, :_N]
    o_ref[...] = jnp.maximum(_dot(t, w_ref[...]) + b_ref[...], 0.0)


def _tc_layer(x, s, W, b2d):
    return pl.pallas_call(
        _tc_layer_body,
        out_shape=jax.ShapeDtypeStruct((_N, _D), jnp.float32),
    )(x, s, W, b2d)


def _tc_stage2_body(h1_ref, s_ref, x_ref, batch_ref, prot_ref, w2_ref, b2_ref,
                    wm1_ref, bm1_ref, wm2_ref, bm2_ref,
                    h2_ref, nb_ref, y_ref):
    h2 = jnp.maximum(_dot(h1_ref[...] + s_ref[0, :_N] + s_ref[1, :_N], w2_ref[...])
                     + b2_ref[...], 0.0)
    h2_ref[...] = h2

    gi = lax.broadcasted_iota(jnp.int32, (1, _G), 1)
    oh = (batch_ref[...] == gi).astype(jnp.float32)
    ge = _dot_tn(oh, h2)

    prot = prot_ref[...]
    gn = _rownorm(ge)
    pn = _rownorm(prot)
    sim0 = _dot_t(ge, prot) / _dot_t(gn, pn)

    mx = jnp.max(sim0, axis=1, keepdims=True)
    pi = lax.broadcasted_iota(jnp.int32, (_G, _P), 1)
    assign = jnp.min(jnp.where(sim0 >= mx, pi, _P), axis=1, keepdims=True)
    oh_a = (assign == pi).astype(jnp.float32)
    p_assigned = _dot(oh_a, prot)

    wm1 = wm1_ref[...]
    pergraph = _dot(p_assigned, wm1[_D:, :])
    t = jnp.maximum(_dot(h2, wm1[:_D, :]) + _dot(oh, pergraph) + bm1_ref[...],
                    0.0)
    prob = _dot(t, wm2_ref[...]) + bm2_ref[...]
    nb = jax.nn.sigmoid(prob)
    nb_ref[...] = nb
    y_ref[...] = x_ref[...] * (nb * nb)


def _tc_stage2(h1, s2, x, batch2d, prot, W2, b2d, Wm1, bm1d, Wm2, bm2d):
    return pl.pallas_call(
        _tc_stage2_body,
        out_shape=[
            jax.ShapeDtypeStruct((_N, _D), jnp.float32),
            jax.ShapeDtypeStruct((_N, 1), jnp.float32),
            jax.ShapeDtypeStruct((_N, _D), jnp.float32),
        ],
    )(h1, s2, x, batch2d, prot, W2, b2d, Wm1, bm1d, Wm2, bm2d)


def _tc_stage3_body(x_ref, s_ref, nb_ref, w1_ref, b1_ref, g1_ref, z_ref):
    nb = nb_ref[...]
    t = nb * (x_ref[...] + s_ref[0, :_N] + s_ref[1, :_N])
    g1 = jnp.maximum(_dot(t, w1_ref[...]) + b1_ref[...], 0.0)
    g1_ref[...] = g1
    z_ref[...] = g1 * nb


def _tc_stage3(x, s3, nb, W1, b1d):
    return pl.pallas_call(
        _tc_stage3_body,
        out_shape=[
            jax.ShapeDtypeStruct((_N, _D), jnp.float32),
            jax.ShapeDtypeStruct((_N, _D), jnp.float32),
        ],
    )(x, s3, nb, W1, b1d)


def _tc_stage4_body(g1_ref, s_ref, nb_ref, eb_ref, batch_ref, prot_ref,
                    w2_ref, b2_ref,
                    se_ref, sim_ref, dsim_ref, kl_ref, nce_ref):
    nb = nb_ref[...]
    t = g1_ref[...] + nb * (s_ref[0, :_N] + s_ref[1, :_N])
    g2 = jnp.maximum(_dot(t, w2_ref[...]) + b2_ref[...], 0.0)

    gi = lax.broadcasted_iota(jnp.int32, (1, _G), 1)
    oh = (batch_ref[...] == gi).astype(jnp.float32)
    se = _dot_tn(oh, g2)
    se_ref[...] = se

    prot = prot_ref[...]
    sn = _rownorm(se)
    pn = _rownorm(prot)
    sim = _dot_t(se, prot) / _dot_t(sn, pn)
    sim_ref[...] = sim
    dsim_ref[...] = _dot_t(se, se) / _dot_t(sn, sn)

    mx = jnp.max(sim, axis=1, keepdims=True)
    pi = lax.broadcasted_iota(jnp.int32, (_G, _P), 1)
    assign = jnp.min(jnp.where(sim >= mx, pi, _P), axis=1, keepdims=True)
    oh_a = (assign == pi).astype(jnp.float32)
    s = jnp.exp(sim * 5.0)
    pos = jnp.sum(s * oh_a, axis=1, keepdims=True)
    neg = jnp.sum(s * (1.0 - oh_a), axis=1, keepdims=True)
    nce = -jnp.mean(jnp.log(pos / neg))
    nce_ref[...] = nce.reshape(1, 1)

    kn = jnp.mean(nb * jnp.log(nb / _R + _EPS)
                  + (1.0 - nb) * jnp.log((1.0 - nb) / (1.0 - _R + _EPS) + _EPS))
    eb = eb_ref[...]
    rr = _R * _R
    ke = jnp.mean(eb * jnp.log(eb / rr + _EPS)
                  + (1.0 - eb) * jnp.log((1.0 - eb) / (1.0 - rr + _EPS) + _EPS))
    kl_ref[...] = (kn + ke).reshape(1, 1)


def _tc_stage4(g1, s4, nb, eb2d, batch2d, prot, W2, b2d):
    return pl.pallas_call(
        _tc_stage4_body,
        out_shape=[
            jax.ShapeDtypeStruct((_G, _D), jnp.float32),
            jax.ShapeDtypeStruct((_G, _P), jnp.float32),
            jax.ShapeDtypeStruct((_G, _G), jnp.float32),
            jax.ShapeDtypeStruct((1, 1), jnp.float32),
            jax.ShapeDtypeStruct((1, 1), jnp.float32),
        ],
    )(g1, s4, nb, eb2d, batch2d, prot, W2, b2d)


def kernel(x, edge_index, batch, W1, b1, W2, b2, Wm1, bm1, Wm2, bm2,
           prototypes):
    src = edge_index[0]
    dst = edge_index[1]
    src3d = src.reshape(_NW, _NCH2, _K)
    dst4d = dst.reshape(_NW, _NB2, _BC2, _K)
    batch2d = batch.reshape(_N, 1)
    b1d = b1.reshape(1, _D)
    b2d = b2.reshape(1, _D)
    bm1d = bm1.reshape(1, _D)
    bm2d = bm2.reshape(1, 1)

    s1 = _sc_segsum(x, src3d, dst4d)
    h1 = _tc_layer(x, s1, W1, b1d)
    s2 = _sc_segsum(h1, src3d, dst4d)
    h2, nb, y = _tc_stage2(h1, s2, x, batch2d, prototypes, W2, b2d,
                           Wm1, bm1d, Wm2, bm2d)
    nbrep = jnp.broadcast_to(nb, (_N, _D))
    eb = _sc_edge_bern(nbrep, src.reshape(_NW, _EPT // _K, 1, _K),
                       dst.reshape(_NW, _EPT // _K, 1, _K))
    eb = eb.reshape(_E, 16)[:, 0]
    s3 = _sc_segsum(y, src3d, dst4d)
    g1, z = _tc_stage3(x, s3, nb, W1, b1d)
    s4 = _sc_segsum(z, src3d, dst4d)
    se, sim, dsim, kl, nce = _tc_stage4(g1, s4, nb, eb.reshape(_E // _D, _D),
                                        batch2d, prototypes, W2, b2d)

    return (kl[0, 0], nce[0, 0], sim, nb, eb.reshape(_E, 1), dsim, se, h2)


# speedup vs baseline: 14.4881x; 1.0986x over previous
import functools

import jax
import jax.numpy as jnp
from jax import lax
from jax.experimental import pallas as pl
from jax.experimental.pallas import tpu as pltpu
from jax.experimental.pallas import tpu_sc as plsc

_N = 10000
_E = 320000
_D = 128
_G = 128
_P = 16
_EPS = 1e-07
_R = 0.5

_NC = 2
_NS = 16
_NW = _NC * _NS
_EPT = _E // _NW
_ESS = _E // _NS
_K = 100

_EPW = _E // _NW
_NCH2 = _EPW // _K
_BC2 = 5
_NB2 = _NCH2 // _BC2
_NP = 10240
_RPT = _NP // _NS


def _segsum_body(table_hbm, src_hbm, dst_hbm, out_hbm,
                 src_v, dstb, rows_v, zbuf, acc, semi0, semi1, semg0, semg1):
    cid = lax.axis_index("c")
    sid = lax.axis_index("s")
    wid = sid * _NC + cid
    semis = (semi0, semi1)
    semgs = (semg0, semg1)
    row0 = sid * _RPT

    pltpu.sync_copy(src_hbm.at[wid], src_v)

    def _zrow(i, c):
        for j in range(_D // 16):
            zbuf[i, pl.ds(j * 16, 16)] = jnp.zeros((16,), jnp.float32)
        return c
    lax.fori_loop(0, 8, _zrow, None)
    for q in range(_RPT // 8):
        pltpu.sync_copy(zbuf, acc.at[pl.ds(row0 + q * 8, 8)])

    plsc.subcore_barrier()

    def _g_start(ci, b):
        pltpu.make_async_copy(
            table_hbm.at[src_v.at[ci]], rows_v.at[b], semgs[b]).start()

    def _g_wait(ci, b):
        pltpu.make_async_copy(
            table_hbm.at[src_v.at[ci]], rows_v.at[b], semgs[b]).wait()

    def _idx_start(blk, s):
        pltpu.make_async_copy(
            dst_hbm.at[wid, blk], dstb.at[s], semis[s]).start()

    def _idx_wait(blk, s):
        pltpu.make_async_copy(
            dst_hbm.at[wid, blk], dstb.at[s], semis[s]).wait()

    _idx_start(0, 0)
    _idx_start(1, 1)
    _g_start(0, 0)
    _g_start(1, 1)

    def _block(blk, s, par):
        _idx_wait(blk, s)
        for j in range(_BC2):
            ci = blk * _BC2 + j
            b = (par + j) % 2
            _g_wait(ci, b)
            pltpu.sync_copy(rows_v.at[b], acc.at[dstb.at[s, j]], add=True)
            nxt = ci + 2

            @pl.when(nxt < _NCH2)
            def _():
                _g_start(nxt, b)

    def _blockpair(t, c):
        for sblk in range(2):
            blk = 2 * t + sblk
            _block(blk, sblk, sblk)
            nxtb = blk + 2

            @pl.when(nxtb < _NB2)
            def _():
                _idx_start(nxtb, sblk)
        return c
    lax.fori_loop(0, _NB2 // 2, _blockpair, None)

    plsc.subcore_barrier()
    pltpu.sync_copy(acc.at[pl.ds(row0, _RPT)],
                    out_hbm.at[cid, pl.ds(row0, _RPT)])


def _sc_segsum(table, src3d, dst4d):
    kern = pl.kernel(
        _segsum_body,
        out_type=jax.ShapeDtypeStruct((_NC, _NP, _D), jnp.float32),
        mesh=plsc.VectorSubcoreMesh(core_axis_name="c", subcore_axis_name="s"),
        scratch_types=[
            pltpu.VMEM((_NCH2, _K), jnp.int32),
            pltpu.VMEM((2, _BC2, _K), jnp.int32),
            pltpu.VMEM((2, _K, _D), jnp.float32),
            pltpu.VMEM((8, _D), jnp.float32),
            pltpu.VMEM_SHARED((_NP, _D), jnp.float32),
            pltpu.SemaphoreType.DMA,
            pltpu.SemaphoreType.DMA,
            pltpu.SemaphoreType.DMA,
            pltpu.SemaphoreType.DMA,
        ],
    )
    return kern(table, src3d, dst4d)


def _edge_bern_body(nb_hbm, src_hbm, dst_hbm, out_hbm,
                    si0, si1, di0, di1, a0, a1, b0, b1, o0, o1,
                    semi0, semi1, sg0, sg1, semo0, semo1):
    cid = lax.axis_index("c")
    sid = lax.axis_index("s")
    wid = sid * _NC + cid
    si = (si0, si1)
    di = (di0, di1)
    av = (a0, a1)
    bv = (b0, b1)
    ov = (o0, o1)
    semi = (semi0, semi1)
    sg = (sg0, sg1)
    semo = (semo0, semo1)
    nch = _EPT // _K

    def _fire_idx(ci, s):
        pltpu.make_async_copy(src_hbm.at[wid, ci], si[s], semi[s]).start()
        pltpu.make_async_copy(dst_hbm.at[wid, ci], di[s], semi[s]).start()

    def _drain_idx(ci, s):
        pltpu.make_async_copy(src_hbm.at[wid, ci], si[s], semi[s]).wait()
        pltpu.make_async_copy(dst_hbm.at[wid, ci], di[s], semi[s]).wait()

    def _fire_g(s):
        pltpu.make_async_copy(nb_hbm.at[si[s].at[0]], av[s], sg[s]).start()
        pltpu.make_async_copy(nb_hbm.at[di[s].at[0]], bv[s], sg[s]).start()

    def _drain_g(s):
        pltpu.make_async_copy(nb_hbm.at[si[s].at[0]], av[s], sg[s]).wait()
        pltpu.make_async_copy(nb_hbm.at[di[s].at[0]], bv[s], sg[s]).wait()

    _fire_idx(0, 0)
    _drain_idx(0, 0)
    _fire_g(0)
    _fire_idx(1, 1)
    for ci in range(nch):
        s = ci % 2
        if ci + 1 < nch:
            _drain_idx(ci + 1, 1 - s)
            _fire_g(1 - s)
        _drain_g(s)
        if ci + 2 < nch:
            _fire_idx(ci + 2, s)

        if ci >= 2:
            pltpu.make_async_copy(ov[s], out_hbm.at[wid, ci - 2],
                                  semo[s]).wait()

        def _edge(r, c):
            ov[s][r] = av[s][r, pl.ds(0, 16)] * bv[s][r, pl.ds(0, 16)]
            return c
        lax.fori_loop(0, _K, _edge, None)
        pltpu.make_async_copy(ov[s], out_hbm.at[wid, ci], semo[s]).start()
    pltpu.make_async_copy(ov[0], out_hbm.at[wid, nch - 2], semo[0]).wait()
    pltpu.make_async_copy(ov[1], out_hbm.at[wid, nch - 1], semo[1]).wait()


def _sc_edge_bern(nbrep, src4, dst4):
    kern = pl.kernel(
        _edge_bern_body,
        out_type=jax.ShapeDtypeStruct((_NW, _EPT // _K, _K, 16), jnp.float32),
        mesh=plsc.VectorSubcoreMesh(core_axis_name="c", subcore_axis_name="s"),
        scratch_types=[
            pltpu.VMEM((1, _K), jnp.int32),
            pltpu.VMEM((1, _K), jnp.int32),
            pltpu.VMEM((1, _K), jnp.int32),
            pltpu.VMEM((1, _K), jnp.int32),
            pltpu.VMEM((_K, _D), jnp.float32),
            pltpu.VMEM((_K, _D), jnp.float32),
            pltpu.VMEM((_K, _D), jnp.float32),
            pltpu.VMEM((_K, _D), jnp.float32),
            pltpu.VMEM((_K, 16), jnp.float32),
            pltpu.VMEM((_K, 16), jnp.float32),
            pltpu.SemaphoreType.DMA,
            pltpu.SemaphoreType.DMA,
            pltpu.SemaphoreType.DMA,
            pltpu.SemaphoreType.DMA,
            pltpu.SemaphoreType.DMA,
            pltpu.SemaphoreType.DMA,
        ],
    )
    return kern(nbrep, src4, dst4)


def _dot(a, b):
    return jax.lax.dot_general(a, b, (((1,), (0,)), ((), ())),
                               preferred_element_type=jnp.float32)


def _dot_t(a, b):
    return jax.lax.dot_general(a, b, (((1,), (1,)), ((), ())),
                               preferred_element_type=jnp.float32)


def _dot_tn(a, b):
    return jax.lax.dot_general(a, b, (((0,), (0,)), ((), ())),
                               preferred_element_type=jnp.float32)


def _rownorm(a):
    n = jnp.sqrt(jnp.sum(a * a, axis=1, keepdims=True))
    return jnp.where(n == 0.0, n + _EPS, n)


def _tc_layer_body(x_ref, s_ref, w_ref, b_ref, o_ref):
    t = x_ref[...] + s_ref[0, :_N] + s_ref[1, :_N]
    o_ref[...] = jnp.maximum(_dot(t, w_ref[...]) + b_ref[...], 0.0)


def _tc_layer(x, s, W, b2d):
    return pl.pallas_call(
        _tc_layer_body,
        out_shape=jax.ShapeDtypeStruct((_N, _D), jnp.float32),
    )(x, s, W, b2d)


def _tc_stage2_body(h1_ref, s_ref, x_ref, batch_ref, prot_ref, w2_ref, b2_ref,
                    wm1_ref, bm1_ref, wm2_ref, bm2_ref,
                    h2_ref, nb_ref, y_ref):
    h2 = jnp.maximum(_dot(h1_ref[...] + s_ref[0, :_N] + s_ref[1, :_N], w2_ref[...])
                     + b2_ref[...], 0.0)
    h2_ref[...] = h2

    gi = lax.broadcasted_iota(jnp.int32, (1, _G), 1)
    oh = (batch_ref[...] == gi).astype(jnp.float32)
    ge = _dot_tn(oh, h2)

    prot = prot_ref[...]
    gn = _rownorm(ge)
    pn = _rownorm(prot)
    sim0 = _dot_t(ge, prot) / _dot_t(gn, pn)

    mx = jnp.max(sim0, axis=1, keepdims=True)
    pi = lax.broadcasted_iota(jnp.int32, (_G, _P), 1)
    assign = jnp.min(jnp.where(sim0 >= mx, pi, _P), axis=1, keepdims=True)
    oh_a = (assign == pi).astype(jnp.float32)
    p_assigned = _dot(oh_a, prot)

    wm1 = wm1_ref[...]
    pergraph = _dot(p_assigned, wm1[_D:, :])
    t = jnp.maximum(_dot(h2, wm1[:_D, :]) + _dot(oh, pergraph) + bm1_ref[...],
                    0.0)
    prob = _dot(t, wm2_ref[...]) + bm2_ref[...]
    nb = jax.nn.sigmoid(prob)
    nb_ref[...] = nb
    y_ref[...] = x_ref[...] * (nb * nb)


def _tc_stage2(h1, s2, x, batch2d, prot, W2, b2d, Wm1, bm1d, Wm2, bm2d):
    return pl.pallas_call(
        _tc_stage2_body,
        out_shape=[
            jax.ShapeDtypeStruct((_N, _D), jnp.float32),
            jax.ShapeDtypeStruct((_N, 1), jnp.float32),
            jax.ShapeDtypeStruct((_N, _D), jnp.float32),
        ],
    )(h1, s2, x, batch2d, prot, W2, b2d, Wm1, bm1d, Wm2, bm2d)


def _tc_stage3_body(x_ref, s_ref, nb_ref, w1_ref, b1_ref, g1_ref, z_ref):
    nb = nb_ref[...]
    t = nb * (x_ref[...] + s_ref[0, :_N] + s_ref[1, :_N])
    g1 = jnp.maximum(_dot(t, w1_ref[...]) + b1_ref[...], 0.0)
    g1_ref[...] = g1
    z_ref[...] = g1 * nb


def _tc_stage3(x, s3, nb, W1, b1d):
    return pl.pallas_call(
        _tc_stage3_body,
        out_shape=[
            jax.ShapeDtypeStruct((_N, _D), jnp.float32),
            jax.ShapeDtypeStruct((_N, _D), jnp.float32),
        ],
    )(x, s3, nb, W1, b1d)


def _tc_stage4_body(g1_ref, s_ref, nb_ref, eb_ref, batch_ref, prot_ref,
                    w2_ref, b2_ref,
                    se_ref, sim_ref, dsim_ref, kl_ref, nce_ref):
    nb = nb_ref[...]
    t = g1_ref[...] + nb * (s_ref[0, :_N] + s_ref[1, :_N])
    g2 = jnp.maximum(_dot(t, w2_ref[...]) + b2_ref[...], 0.0)

    gi = lax.broadcasted_iota(jnp.int32, (1, _G), 1)
    oh = (batch_ref[...] == gi).astype(jnp.float32)
    se = _dot_tn(oh, g2)
    se_ref[...] = se

    prot = prot_ref[...]
    sn = _rownorm(se)
    pn = _rownorm(prot)
    sim = _dot_t(se, prot) / _dot_t(sn, pn)
    sim_ref[...] = sim
    dsim_ref[...] = _dot_t(se, se) / _dot_t(sn, sn)

    mx = jnp.max(sim, axis=1, keepdims=True)
    pi = lax.broadcasted_iota(jnp.int32, (_G, _P), 1)
    assign = jnp.min(jnp.where(sim >= mx, pi, _P), axis=1, keepdims=True)
    oh_a = (assign == pi).astype(jnp.float32)
    s = jnp.exp(sim * 5.0)
    pos = jnp.sum(s * oh_a, axis=1, keepdims=True)
    neg = jnp.sum(s * (1.0 - oh_a), axis=1, keepdims=True)
    nce = -jnp.mean(jnp.log(pos / neg))
    nce_ref[...] = nce.reshape(1, 1)

    kn = jnp.mean(nb * jnp.log(nb / _R + _EPS)
                  + (1.0 - nb) * jnp.log((1.0 - nb) / (1.0 - _R + _EPS) + _EPS))
    eb = eb_ref[...]
    rr = _R * _R
    ke = jnp.mean(eb * jnp.log(eb / rr + _EPS)
                  + (1.0 - eb) * jnp.log((1.0 - eb) / (1.0 - rr + _EPS) + _EPS))
    kl_ref[...] = (kn + ke).reshape(1, 1)


def _tc_stage4(g1, s4, nb, eb2d, batch2d, prot, W2, b2d):
    return pl.pallas_call(
        _tc_stage4_body,
        out_shape=[
            jax.ShapeDtypeStruct((_G, _D), jnp.float32),
            jax.ShapeDtypeStruct((_G, _P), jnp.float32),
            jax.ShapeDtypeStruct((_G, _G), jnp.float32),
            jax.ShapeDtypeStruct((1, 1), jnp.float32),
            jax.ShapeDtypeStruct((1, 1), jnp.float32),
        ],
    )(g1, s4, nb, eb2d, batch2d, prot, W2, b2d)


def kernel(x, edge_index, batch, W1, b1, W2, b2, Wm1, bm1, Wm2, bm2,
           prototypes):
    src = edge_index[0]
    dst = edge_index[1]
    src3d = src.reshape(_NW, _NCH2, _K)
    dst4d = dst.reshape(_NW, _NB2, _BC2, _K)
    batch2d = batch.reshape(_N, 1)
    b1d = b1.reshape(1, _D)
    b2d = b2.reshape(1, _D)
    bm1d = bm1.reshape(1, _D)
    bm2d = bm2.reshape(1, 1)

    s1 = _sc_segsum(x, src3d, dst4d)
    h1 = _tc_layer(x, s1, W1, b1d)
    s2 = _sc_segsum(h1, src3d, dst4d)
    h2, nb, y = _tc_stage2(h1, s2, x, batch2d, prototypes, W2, b2d,
                           Wm1, bm1d, Wm2, bm2d)
    nbrep = jnp.broadcast_to(nb, (_N, _D))
    eb = _sc_edge_bern(nbrep, src.reshape(_NW, _EPT // _K, 1, _K),
                       dst.reshape(_NW, _EPT // _K, 1, _K))
    eb = eb.reshape(_E, 16)[:, 0]
    s3 = _sc_segsum(y, src3d, dst4d)
    g1, z = _tc_stage3(x, s3, nb, W1, b1d)
    s4 = _sc_segsum(z, src3d, dst4d)
    se, sim, dsim, kl, nce = _tc_stage4(g1, s4, nb, eb.reshape(_E // _D, _D),
                                        batch2d, prototypes, W2, b2d)

    return (kl[0, 0], nce[0, 0], sim, nb, eb.reshape(_E, 1), dsim, se, h2)


# speedup vs baseline: 14.6786x; 1.0132x over previous
import functools

import jax
import jax.numpy as jnp
from jax import lax
from jax.experimental import pallas as pl
from jax.experimental.pallas import tpu as pltpu
from jax.experimental.pallas import tpu_sc as plsc

_N = 10000
_E = 320000
_D = 128
_G = 128
_P = 16
_EPS = 1e-07
_R = 0.5

_NC = 2
_NS = 16
_NW = _NC * _NS
_EPT = _E // _NW
_ESS = _E // _NS
_K = 100

_EPW = _E // _NW
_NCH2 = _EPW // _K
_BC2 = 5
_NB2 = _NCH2 // _BC2
_NP = 10240
_RPT = _NP // _NS


def _segsum_body(table_hbm, src_hbm, dst_hbm, out_hbm,
                 src_v, dstb, rows_v, zbuf, acc, semi0, semi1, semg0, semg1):
    cid = lax.axis_index("c")
    sid = lax.axis_index("s")
    wid = sid * _NC + cid
    semis = (semi0, semi1)
    semgs = (semg0, semg1)
    row0 = sid * _RPT

    pltpu.sync_copy(src_hbm.at[wid], src_v)

    def _zrow(i, c):
        for j in range(_D // 16):
            zbuf[i, pl.ds(j * 16, 16)] = jnp.zeros((16,), jnp.float32)
        return c
    lax.fori_loop(0, 8, _zrow, None)
    for q in range(_RPT // 8):
        pltpu.make_async_copy(zbuf, acc.at[pl.ds(row0 + q * 8, 8)],
                              semi0).start()
    for q in range(_RPT // 8):
        pltpu.make_async_copy(zbuf, acc.at[pl.ds(row0 + q * 8, 8)],
                              semi0).wait()

    plsc.subcore_barrier()

    def _g_start(ci, b):
        pltpu.make_async_copy(
            table_hbm.at[src_v.at[ci]], rows_v.at[b], semgs[b]).start()

    def _g_wait(ci, b):
        pltpu.make_async_copy(
            table_hbm.at[src_v.at[ci]], rows_v.at[b], semgs[b]).wait()

    def _idx_start(blk, s):
        pltpu.make_async_copy(
            dst_hbm.at[wid, blk], dstb.at[s], semis[s]).start()

    def _idx_wait(blk, s):
        pltpu.make_async_copy(
            dst_hbm.at[wid, blk], dstb.at[s], semis[s]).wait()

    _idx_start(0, 0)
    _idx_start(1, 1)
    _g_start(0, 0)
    _g_start(1, 1)

    def _block(blk, s, par):
        _idx_wait(blk, s)
        for j in range(_BC2):
            ci = blk * _BC2 + j
            b = (par + j) % 2
            _g_wait(ci, b)
            pltpu.sync_copy(rows_v.at[b], acc.at[dstb.at[s, j]], add=True)
            nxt = ci + 2

            @pl.when(nxt < _NCH2)
            def _():
                _g_start(nxt, b)

    def _blockpair(t, c):
        for sblk in range(2):
            blk = 2 * t + sblk
            _block(blk, sblk, sblk)
            nxtb = blk + 2

            @pl.when(nxtb < _NB2)
            def _():
                _idx_start(nxtb, sblk)
        return c
    lax.fori_loop(0, _NB2 // 2, _blockpair, None)

    plsc.subcore_barrier()
    pltpu.sync_copy(acc.at[pl.ds(row0, _RPT)],
                    out_hbm.at[cid, pl.ds(row0, _RPT)])


def _sc_segsum(table, src3d, dst4d):
    kern = pl.kernel(
        _segsum_body,
        out_type=jax.ShapeDtypeStruct((_NC, _NP, _D), jnp.float32),
        mesh=plsc.VectorSubcoreMesh(core_axis_name="c", subcore_axis_name="s"),
        scratch_types=[
            pltpu.VMEM((_NCH2, _K), jnp.int32),
            pltpu.VMEM((2, _BC2, _K), jnp.int32),
            pltpu.VMEM((2, _K, _D), jnp.float32),
            pltpu.VMEM((8, _D), jnp.float32),
            pltpu.VMEM_SHARED((_NP, _D), jnp.float32),
            pltpu.SemaphoreType.DMA,
            pltpu.SemaphoreType.DMA,
            pltpu.SemaphoreType.DMA,
            pltpu.SemaphoreType.DMA,
        ],
    )
    return kern(table, src3d, dst4d)


def _edge_bern_body(nb_hbm, src_hbm, dst_hbm, out_hbm,
                    si0, si1, di0, di1, a0, a1, b0, b1, o0, o1,
                    semi0, semi1, sg0, sg1, semo0, semo1):
    cid = lax.axis_index("c")
    sid = lax.axis_index("s")
    wid = sid * _NC + cid
    si = (si0, si1)
    di = (di0, di1)
    av = (a0, a1)
    bv = (b0, b1)
    ov = (o0, o1)
    semi = (semi0, semi1)
    sg = (sg0, sg1)
    semo = (semo0, semo1)
    nch = _EPT // _K

    def _fire_idx(ci, s):
        pltpu.make_async_copy(src_hbm.at[wid, ci], si[s], semi[s]).start()
        pltpu.make_async_copy(dst_hbm.at[wid, ci], di[s], semi[s]).start()

    def _drain_idx(ci, s):
        pltpu.make_async_copy(src_hbm.at[wid, ci], si[s], semi[s]).wait()
        pltpu.make_async_copy(dst_hbm.at[wid, ci], di[s], semi[s]).wait()

    def _fire_g(s):
        pltpu.make_async_copy(nb_hbm.at[si[s].at[0]], av[s], sg[s]).start()
        pltpu.make_async_copy(nb_hbm.at[di[s].at[0]], bv[s], sg[s]).start()

    def _drain_g(s):
        pltpu.make_async_copy(nb_hbm.at[si[s].at[0]], av[s], sg[s]).wait()
        pltpu.make_async_copy(nb_hbm.at[di[s].at[0]], bv[s], sg[s]).wait()

    _fire_idx(0, 0)
    _drain_idx(0, 0)
    _fire_g(0)
    _fire_idx(1, 1)
    for ci in range(nch):
        s = ci % 2
        if ci + 1 < nch:
            _drain_idx(ci + 1, 1 - s)
            _fire_g(1 - s)
        _drain_g(s)
        if ci + 2 < nch:
            _fire_idx(ci + 2, s)

        if ci >= 2:
            pltpu.make_async_copy(ov[s], out_hbm.at[wid, ci - 2],
                                  semo[s]).wait()

        def _edge(r, c):
            ov[s][r] = av[s][r, pl.ds(0, 16)] * bv[s][r, pl.ds(0, 16)]
            return c
        lax.fori_loop(0, _K, _edge, None)
        pltpu.make_async_copy(ov[s], out_hbm.at[wid, ci], semo[s]).start()
    pltpu.make_async_copy(ov[0], out_hbm.at[wid, nch - 2], semo[0]).wait()
    pltpu.make_async_copy(ov[1], out_hbm.at[wid, nch - 1], semo[1]).wait()


def _sc_edge_bern(nbrep, src4, dst4):
    kern = pl.kernel(
        _edge_bern_body,
        out_type=jax.ShapeDtypeStruct((_NW, _EPT // _K, _K, 16), jnp.float32),
        mesh=plsc.VectorSubcoreMesh(core_axis_name="c", subcore_axis_name="s"),
        scratch_types=[
            pltpu.VMEM((1, _K), jnp.int32),
            pltpu.VMEM((1, _K), jnp.int32),
            pltpu.VMEM((1, _K), jnp.int32),
            pltpu.VMEM((1, _K), jnp.int32),
            pltpu.VMEM((_K, _D), jnp.float32),
            pltpu.VMEM((_K, _D), jnp.float32),
            pltpu.VMEM((_K, _D), jnp.float32),
            pltpu.VMEM((_K, _D), jnp.float32),
            pltpu.VMEM((_K, 16), jnp.float32),
            pltpu.VMEM((_K, 16), jnp.float32),
            pltpu.SemaphoreType.DMA,
            pltpu.SemaphoreType.DMA,
            pltpu.SemaphoreType.DMA,
            pltpu.SemaphoreType.DMA,
            pltpu.SemaphoreType.DMA,
            pltpu.SemaphoreType.DMA,
        ],
    )
    return kern(nbrep, src4, dst4)


def _dot(a, b):
    return jax.lax.dot_general(a, b, (((1,), (0,)), ((), ())),
                               preferred_element_type=jnp.float32)


def _dot_t(a, b):
    return jax.lax.dot_general(a, b, (((1,), (1,)), ((), ())),
                               preferred_element_type=jnp.float32)


def _dot_tn(a, b):
    return jax.lax.dot_general(a, b, (((0,), (0,)), ((), ())),
                               preferred_element_type=jnp.float32)


def _rownorm(a):
    n = jnp.sqrt(jnp.sum(a * a, axis=1, keepdims=True))
    return jnp.where(n == 0.0, n + _EPS, n)


def _tc_layer_body(x_ref, s_ref, w_ref, b_ref, o_ref):
    t = x_ref[...] + s_ref[0, :_N] + s_ref[1, :_N]
    o_ref[...] = jnp.maximum(_dot(t, w_ref[...]) + b_ref[...], 0.0)


def _tc_layer(x, s, W, b2d):
    return pl.pallas_call(
        _tc_layer_body,
        out_shape=jax.ShapeDtypeStruct((_N, _D), jnp.float32),
    )(x, s, W, b2d)


def _tc_stage2_body(h1_ref, s_ref, x_ref, batch_ref, prot_ref, w2_ref, b2_ref,
                    wm1_ref, bm1_ref, wm2_ref, bm2_ref,
                    h2_ref, nb_ref, y_ref):
    h2 = jnp.maximum(_dot(h1_ref[...] + s_ref[0, :_N] + s_ref[1, :_N], w2_ref[...])
                     + b2_ref[...], 0.0)
    h2_ref[...] = h2

    gi = lax.broadcasted_iota(jnp.int32, (1, _G), 1)
    oh = (batch_ref[...] == gi).astype(jnp.float32)
    ge = _dot_tn(oh, h2)

    prot = prot_ref[...]
    gn = _rownorm(ge)
    pn = _rownorm(prot)
    sim0 = _dot_t(ge, prot) / _dot_t(gn, pn)

    mx = jnp.max(sim0, axis=1, keepdims=True)
    pi = lax.broadcasted_iota(jnp.int32, (_G, _P), 1)
    assign = jnp.min(jnp.where(sim0 >= mx, pi, _P), axis=1, keepdims=True)
    oh_a = (assign == pi).astype(jnp.float32)
    p_assigned = _dot(oh_a, prot)

    wm1 = wm1_ref[...]
    pergraph = _dot(p_assigned, wm1[_D:, :])
    t = jnp.maximum(_dot(h2, wm1[:_D, :]) + _dot(oh, pergraph) + bm1_ref[...],
                    0.0)
    prob = _dot(t, wm2_ref[...]) + bm2_ref[...]
    nb = jax.nn.sigmoid(prob)
    nb_ref[...] = nb
    y_ref[...] = x_ref[...] * (nb * nb)


def _tc_stage2(h1, s2, x, batch2d, prot, W2, b2d, Wm1, bm1d, Wm2, bm2d):
    return pl.pallas_call(
        _tc_stage2_body,
        out_shape=[
            jax.ShapeDtypeStruct((_N, _D), jnp.float32),
            jax.ShapeDtypeStruct((_N, 1), jnp.float32),
            jax.ShapeDtypeStruct((_N, _D), jnp.float32),
        ],
    )(h1, s2, x, batch2d, prot, W2, b2d, Wm1, bm1d, Wm2, bm2d)


def _tc_stage3_body(x_ref, s_ref, nb_ref, w1_ref, b1_ref, g1_ref, z_ref):
    nb = nb_ref[...]
    t = nb * (x_ref[...] + s_ref[0, :_N] + s_ref[1, :_N])
    g1 = jnp.maximum(_dot(t, w1_ref[...]) + b1_ref[...], 0.0)
    g1_ref[...] = g1
    z_ref[...] = g1 * nb


def _tc_stage3(x, s3, nb, W1, b1d):
    return pl.pallas_call(
        _tc_stage3_body,
        out_shape=[
            jax.ShapeDtypeStruct((_N, _D), jnp.float32),
            jax.ShapeDtypeStruct((_N, _D), jnp.float32),
        ],
    )(x, s3, nb, W1, b1d)


def _tc_stage4_body(g1_ref, s_ref, nb_ref, eb_ref, batch_ref, prot_ref,
                    w2_ref, b2_ref,
                    se_ref, sim_ref, dsim_ref, kl_ref, nce_ref):
    nb = nb_ref[...]
    t = g1_ref[...] + nb * (s_ref[0, :_N] + s_ref[1, :_N])
    g2 = jnp.maximum(_dot(t, w2_ref[...]) + b2_ref[...], 0.0)

    gi = lax.broadcasted_iota(jnp.int32, (1, _G), 1)
    oh = (batch_ref[...] == gi).astype(jnp.float32)
    se = _dot_tn(oh, g2)
    se_ref[...] = se

    prot = prot_ref[...]
    sn = _rownorm(se)
    pn = _rownorm(prot)
    sim = _dot_t(se, prot) / _dot_t(sn, pn)
    sim_ref[...] = sim
    dsim_ref[...] = _dot_t(se, se) / _dot_t(sn, sn)

    mx = jnp.max(sim, axis=1, keepdims=True)
    pi = lax.broadcasted_iota(jnp.int32, (_G, _P), 1)
    assign = jnp.min(jnp.where(sim >= mx, pi, _P), axis=1, keepdims=True)
    oh_a = (assign == pi).astype(jnp.float32)
    s = jnp.exp(sim * 5.0)
    pos = jnp.sum(s * oh_a, axis=1, keepdims=True)
    neg = jnp.sum(s * (1.0 - oh_a), axis=1, keepdims=True)
    nce = -jnp.mean(jnp.log(pos / neg))
    nce_ref[...] = nce.reshape(1, 1)

    kn = jnp.mean(nb * jnp.log(nb / _R + _EPS)
                  + (1.0 - nb) * jnp.log((1.0 - nb) / (1.0 - _R + _EPS) + _EPS))
    eb = eb_ref[...]
    rr = _R * _R
    ke = jnp.mean(eb * jnp.log(eb / rr + _EPS)
                  + (1.0 - eb) * jnp.log((1.0 - eb) / (1.0 - rr + _EPS) + _EPS))
    kl_ref[...] = (kn + ke).reshape(1, 1)


def _tc_stage4(g1, s4, nb, eb2d, batch2d, prot, W2, b2d):
    return pl.pallas_call(
        _tc_stage4_body,
        out_shape=[
            jax.ShapeDtypeStruct((_G, _D), jnp.float32),
            jax.ShapeDtypeStruct((_G, _P), jnp.float32),
            jax.ShapeDtypeStruct((_G, _G), jnp.float32),
            jax.ShapeDtypeStruct((1, 1), jnp.float32),
            jax.ShapeDtypeStruct((1, 1), jnp.float32),
        ],
    )(g1, s4, nb, eb2d, batch2d, prot, W2, b2d)


def kernel(x, edge_index, batch, W1, b1, W2, b2, Wm1, bm1, Wm2, bm2,
           prototypes):
    src = edge_index[0]
    dst = edge_index[1]
    src3d = src.reshape(_NW, _NCH2, _K)
    dst4d = dst.reshape(_NW, _NB2, _BC2, _K)
    batch2d = batch.reshape(_N, 1)
    b1d = b1.reshape(1, _D)
    b2d = b2.reshape(1, _D)
    bm1d = bm1.reshape(1, _D)
    bm2d = bm2.reshape(1, 1)

    s1 = _sc_segsum(x, src3d, dst4d)
    h1 = _tc_layer(x, s1, W1, b1d)
    s2 = _sc_segsum(h1, src3d, dst4d)
    h2, nb, y = _tc_stage2(h1, s2, x, batch2d, prototypes, W2, b2d,
                           Wm1, bm1d, Wm2, bm2d)
    nbrep = jnp.broadcast_to(nb, (_N, _D))
    eb = _sc_edge_bern(nbrep, src.reshape(_NW, _EPT // _K, 1, _K),
                       dst.reshape(_NW, _EPT // _K, 1, _K))
    eb = eb.reshape(_E, 16)[:, 0]
    s3 = _sc_segsum(y, src3d, dst4d)
    g1, z = _tc_stage3(x, s3, nb, W1, b1d)
    s4 = _sc_segsum(z, src3d, dst4d)
    se, sim, dsim, kl, nce = _tc_stage4(g1, s4, nb, eb.reshape(_E // _D, _D),
                                        batch2d, prototypes, W2, b2d)

    return (kl[0, 0], nce[0, 0], sim, nb, eb.reshape(_E, 1), dsim, se, h2)


# speedup vs baseline: 14.7062x; 1.0019x over previous
import jax
import jax.numpy as jnp
from jax import lax
from jax.experimental import pallas as pl
from jax.experimental.pallas import tpu as pltpu
from jax.experimental.pallas import tpu_sc as plsc

_N = 10000
_E = 320000
_D = 128
_G = 128
_P = 16
_EPS = 1e-07
_R = 0.5

_NC = 2
_NS = 16
_NW = _NC * _NS
_EPT = _E // _NW
_K = 100

_EPW = _E // _NW
_NCH2 = _EPW // _K
_BC2 = 5
_NB2 = _NCH2 // _BC2
_NP = 10240
_RPT = _NP // _NS


def _segsum_body(table_hbm, src_hbm, dst_hbm, out_hbm,
                 src_v, dstb, rows_v, zbuf, acc, semi0, semi1, semg0, semg1):
    cid = lax.axis_index("c")
    sid = lax.axis_index("s")
    wid = sid * _NC + cid
    semis = (semi0, semi1)
    semgs = (semg0, semg1)
    row0 = sid * _RPT

    pltpu.sync_copy(src_hbm.at[wid], src_v)

    def _zrow(i, c):
        for j in range(_D // 16):
            zbuf[i, pl.ds(j * 16, 16)] = jnp.zeros((16,), jnp.float32)
        return c
    lax.fori_loop(0, 8, _zrow, None)
    for q in range(_RPT // 8):
        pltpu.make_async_copy(zbuf, acc.at[pl.ds(row0 + q * 8, 8)],
                              semi0).start()
    for q in range(_RPT // 8):
        pltpu.make_async_copy(zbuf, acc.at[pl.ds(row0 + q * 8, 8)],
                              semi0).wait()

    plsc.subcore_barrier()

    def _g_start(ci, b):
        pltpu.make_async_copy(
            table_hbm.at[src_v.at[ci]], rows_v.at[b], semgs[b]).start()

    def _g_wait(ci, b):
        pltpu.make_async_copy(
            table_hbm.at[src_v.at[ci]], rows_v.at[b], semgs[b]).wait()

    def _idx_start(blk, s):
        pltpu.make_async_copy(
            dst_hbm.at[wid, blk], dstb.at[s], semis[s]).start()

    def _idx_wait(blk, s):
        pltpu.make_async_copy(
            dst_hbm.at[wid, blk], dstb.at[s], semis[s]).wait()

    _idx_start(0, 0)
    _idx_start(1, 1)
    _g_start(0, 0)
    _g_start(1, 1)

    def _block(blk, s, par):
        _idx_wait(blk, s)
        for j in range(_BC2):
            ci = blk * _BC2 + j
            b = (par + j) % 2
            _g_wait(ci, b)
            pltpu.sync_copy(rows_v.at[b], acc.at[dstb.at[s, j]], add=True)
            nxt = ci + 2

            @pl.when(nxt < _NCH2)
            def _():
                _g_start(nxt, b)

    def _blockpair(t, c):
        for sblk in range(2):
            blk = 2 * t + sblk
            _block(blk, sblk, sblk)
            nxtb = blk + 2

            @pl.when(nxtb < _NB2)
            def _():
                _idx_start(nxtb, sblk)
        return c
    lax.fori_loop(0, _NB2 // 2, _blockpair, None)

    plsc.subcore_barrier()
    pltpu.sync_copy(acc.at[pl.ds(row0, _RPT)],
                    out_hbm.at[cid, pl.ds(row0, _RPT)])


def _sc_segsum(table, src3d, dst4d):
    kern = pl.kernel(
        _segsum_body,
        out_type=jax.ShapeDtypeStruct((_NC, _NP, _D), jnp.float32),
        mesh=plsc.VectorSubcoreMesh(core_axis_name="c", subcore_axis_name="s"),
        scratch_types=[
            pltpu.VMEM((_NCH2, _K), jnp.int32),
            pltpu.VMEM((2, _BC2, _K), jnp.int32),
            pltpu.VMEM((2, _K, _D), jnp.float32),
            pltpu.VMEM((8, _D), jnp.float32),
            pltpu.VMEM_SHARED((_NP, _D), jnp.float32),
            pltpu.SemaphoreType.DMA,
            pltpu.SemaphoreType.DMA,
            pltpu.SemaphoreType.DMA,
            pltpu.SemaphoreType.DMA,
        ],
    )
    return kern(table, src3d, dst4d)


def _edge_bern_body(nb_hbm, src_hbm, dst_hbm, out_hbm,
                    si0, si1, di0, di1, a0, a1, b0, b1, o0, o1,
                    semi0, semi1, sg0, sg1, semo0, semo1):
    cid = lax.axis_index("c")
    sid = lax.axis_index("s")
    wid = sid * _NC + cid
    si = (si0, si1)
    di = (di0, di1)
    av = (a0, a1)
    bv = (b0, b1)
    ov = (o0, o1)
    semi = (semi0, semi1)
    sg = (sg0, sg1)
    semo = (semo0, semo1)
    nch = _EPT // _K

    def _fire_idx(ci, s):
        pltpu.make_async_copy(src_hbm.at[wid, ci], si[s], semi[s]).start()
        pltpu.make_async_copy(dst_hbm.at[wid, ci], di[s], semi[s]).start()

    def _drain_idx(ci, s):
        pltpu.make_async_copy(src_hbm.at[wid, ci], si[s], semi[s]).wait()
        pltpu.make_async_copy(dst_hbm.at[wid, ci], di[s], semi[s]).wait()

    def _fire_g(s):
        pltpu.make_async_copy(nb_hbm.at[si[s].at[0]], av[s], sg[s]).start()
        pltpu.make_async_copy(nb_hbm.at[di[s].at[0]], bv[s], sg[s]).start()

    def _drain_g(s):
        pltpu.make_async_copy(nb_hbm.at[si[s].at[0]], av[s], sg[s]).wait()
        pltpu.make_async_copy(nb_hbm.at[di[s].at[0]], bv[s], sg[s]).wait()

    _fire_idx(0, 0)
    _drain_idx(0, 0)
    _fire_g(0)
    _fire_idx(1, 1)
    for ci in range(nch):
        s = ci % 2
        if ci + 1 < nch:
            _drain_idx(ci + 1, 1 - s)
            _fire_g(1 - s)
        _drain_g(s)
        if ci + 2 < nch:
            _fire_idx(ci + 2, s)

        if ci >= 2:
            pltpu.make_async_copy(ov[s], out_hbm.at[wid, ci - 2],
                                  semo[s]).wait()

        def _edge(r, c):
            ov[s][r] = av[s][r, pl.ds(0, 16)] * bv[s][r, pl.ds(0, 16)]
            return c
        lax.fori_loop(0, _K, _edge, None)
        pltpu.make_async_copy(ov[s], out_hbm.at[wid, ci], semo[s]).start()
    pltpu.make_async_copy(ov[0], out_hbm.at[wid, nch - 2], semo[0]).wait()
    pltpu.make_async_copy(ov[1], out_hbm.at[wid, nch - 1], semo[1]).wait()


def _sc_edge_bern(nbrep, src4, dst4):
    kern = pl.kernel(
        _edge_bern_body,
        out_type=jax.ShapeDtypeStruct((_NW, _EPT // _K, _K, 16), jnp.float32),
        mesh=plsc.VectorSubcoreMesh(core_axis_name="c", subcore_axis_name="s"),
        scratch_types=[
            pltpu.VMEM((1, _K), jnp.int32),
            pltpu.VMEM((1, _K), jnp.int32),
            pltpu.VMEM((1, _K), jnp.int32),
            pltpu.VMEM((1, _K), jnp.int32),
            pltpu.VMEM((_K, _D), jnp.float32),
            pltpu.VMEM((_K, _D), jnp.float32),
            pltpu.VMEM((_K, _D), jnp.float32),
            pltpu.VMEM((_K, _D), jnp.float32),
            pltpu.VMEM((_K, 16), jnp.float32),
            pltpu.VMEM((_K, 16), jnp.float32),
            pltpu.SemaphoreType.DMA,
            pltpu.SemaphoreType.DMA,
            pltpu.SemaphoreType.DMA,
            pltpu.SemaphoreType.DMA,
            pltpu.SemaphoreType.DMA,
            pltpu.SemaphoreType.DMA,
        ],
    )
    return kern(nbrep, src4, dst4)


def _dot(a, b):
    return jax.lax.dot_general(a, b, (((1,), (0,)), ((), ())),
                               preferred_element_type=jnp.float32)


def _dot_t(a, b):
    return jax.lax.dot_general(a, b, (((1,), (1,)), ((), ())),
                               preferred_element_type=jnp.float32)


def _dot_tn(a, b):
    return jax.lax.dot_general(a, b, (((0,), (0,)), ((), ())),
                               preferred_element_type=jnp.float32)


def _rownorm(a):
    n = jnp.sqrt(jnp.sum(a * a, axis=1, keepdims=True))
    return jnp.where(n == 0.0, n + _EPS, n)


def _tc_layer_body(x_ref, s_ref, w_ref, b_ref, o_ref):
    t = x_ref[...] + s_ref[0, :_N] + s_ref[1, :_N]
    o_ref[...] = jnp.maximum(_dot(t, w_ref[...]) + b_ref[...], 0.0)


def _tc_layer(x, s, W, b2d):
    return pl.pallas_call(
        _tc_layer_body,
        out_shape=jax.ShapeDtypeStruct((_N, _D), jnp.float32),
    )(x, s, W, b2d)


def _tc_stage2_body(h1_ref, s_ref, x_ref, batch_ref, prot_ref, w2_ref, b2_ref,
                    wm1_ref, bm1_ref, wm2_ref, bm2_ref,
                    h2_ref, nb_ref, y_ref):
    h2 = jnp.maximum(_dot(h1_ref[...] + s_ref[0, :_N] + s_ref[1, :_N], w2_ref[...])
                     + b2_ref[...], 0.0)
    h2_ref[...] = h2

    gi = lax.broadcasted_iota(jnp.int32, (1, _G), 1)
    oh = (batch_ref[...] == gi).astype(jnp.float32)
    ge = _dot_tn(oh, h2)

    prot = prot_ref[...]
    gn = _rownorm(ge)
    pn = _rownorm(prot)
    sim0 = _dot_t(ge, prot) / _dot_t(gn, pn)

    mx = jnp.max(sim0, axis=1, keepdims=True)
    pi = lax.broadcasted_iota(jnp.int32, (_G, _P), 1)
    assign = jnp.min(jnp.where(sim0 >= mx, pi, _P), axis=1, keepdims=True)
    oh_a = (assign == pi).astype(jnp.float32)
    p_assigned = _dot(oh_a, prot)

    wm1 = wm1_ref[...]
    pergraph = _dot(p_assigned, wm1[_D:, :])
    t = jnp.maximum(_dot(h2, wm1[:_D, :]) + _dot(oh, pergraph) + bm1_ref[...],
                    0.0)
    prob = _dot(t, wm2_ref[...]) + bm2_ref[...]
    nb = jax.nn.sigmoid(prob)
    nb_ref[...] = nb
    y_ref[...] = x_ref[...] * (nb * nb)


def _tc_stage2(h1, s2, x, batch2d, prot, W2, b2d, Wm1, bm1d, Wm2, bm2d):
    return pl.pallas_call(
        _tc_stage2_body,
        out_shape=[
            jax.ShapeDtypeStruct((_N, _D), jnp.float32),
            jax.ShapeDtypeStruct((_N, 1), jnp.float32),
            jax.ShapeDtypeStruct((_N, _D), jnp.float32),
        ],
    )(h1, s2, x, batch2d, prot, W2, b2d, Wm1, bm1d, Wm2, bm2d)


def _tc_stage3_body(x_ref, s_ref, nb_ref, w1_ref, b1_ref, g1_ref, z_ref):
    nb = nb_ref[...]
    t = nb * (x_ref[...] + s_ref[0, :_N] + s_ref[1, :_N])
    g1 = jnp.maximum(_dot(t, w1_ref[...]) + b1_ref[...], 0.0)
    g1_ref[...] = g1
    z_ref[...] = g1 * nb


def _tc_stage3(x, s3, nb, W1, b1d):
    return pl.pallas_call(
        _tc_stage3_body,
        out_shape=[
            jax.ShapeDtypeStruct((_N, _D), jnp.float32),
            jax.ShapeDtypeStruct((_N, _D), jnp.float32),
        ],
    )(x, s3, nb, W1, b1d)


def _tc_stage4_body(g1_ref, s_ref, nb_ref, eb_ref, batch_ref, prot_ref,
                    w2_ref, b2_ref,
                    se_ref, sim_ref, dsim_ref, kl_ref, nce_ref):
    nb = nb_ref[...]
    t = g1_ref[...] + nb * (s_ref[0, :_N] + s_ref[1, :_N])
    g2 = jnp.maximum(_dot(t, w2_ref[...]) + b2_ref[...], 0.0)

    gi = lax.broadcasted_iota(jnp.int32, (1, _G), 1)
    oh = (batch_ref[...] == gi).astype(jnp.float32)
    se = _dot_tn(oh, g2)
    se_ref[...] = se

    prot = prot_ref[...]
    sn = _rownorm(se)
    pn = _rownorm(prot)
    sim = _dot_t(se, prot) / _dot_t(sn, pn)
    sim_ref[...] = sim
    dsim_ref[...] = _dot_t(se, se) / _dot_t(sn, sn)

    mx = jnp.max(sim, axis=1, keepdims=True)
    pi = lax.broadcasted_iota(jnp.int32, (_G, _P), 1)
    assign = jnp.min(jnp.where(sim >= mx, pi, _P), axis=1, keepdims=True)
    oh_a = (assign == pi).astype(jnp.float32)
    s = jnp.exp(sim * 5.0)
    pos = jnp.sum(s * oh_a, axis=1, keepdims=True)
    neg = jnp.sum(s * (1.0 - oh_a), axis=1, keepdims=True)
    nce = -jnp.mean(jnp.log(pos / neg))
    nce_ref[...] = nce.reshape(1, 1)

    kn = jnp.mean(nb * jnp.log(nb / _R + _EPS)
                  + (1.0 - nb) * jnp.log((1.0 - nb) / (1.0 - _R + _EPS) + _EPS))
    eb = eb_ref[...]
    rr = _R * _R
    ke = jnp.mean(eb * jnp.log(eb / rr + _EPS)
                  + (1.0 - eb) * jnp.log((1.0 - eb) / (1.0 - rr + _EPS) + _EPS))
    kl_ref[...] = (kn + ke).reshape(1, 1)


def _tc_stage4(g1, s4, nb, eb2d, batch2d, prot, W2, b2d):
    return pl.pallas_call(
        _tc_stage4_body,
        out_shape=[
            jax.ShapeDtypeStruct((_G, _D), jnp.float32),
            jax.ShapeDtypeStruct((_G, _P), jnp.float32),
            jax.ShapeDtypeStruct((_G, _G), jnp.float32),
            jax.ShapeDtypeStruct((1, 1), jnp.float32),
            jax.ShapeDtypeStruct((1, 1), jnp.float32),
        ],
    )(g1, s4, nb, eb2d, batch2d, prot, W2, b2d)


def kernel(x, edge_index, batch, W1, b1, W2, b2, Wm1, bm1, Wm2, bm2,
           prototypes):
    src = edge_index[0]
    dst = edge_index[1]
    src3d = src.reshape(_NW, _NCH2, _K)
    dst4d = dst.reshape(_NW, _NB2, _BC2, _K)
    batch2d = batch.reshape(_N, 1)
    b1d = b1.reshape(1, _D)
    b2d = b2.reshape(1, _D)
    bm1d = bm1.reshape(1, _D)
    bm2d = bm2.reshape(1, 1)

    s1 = _sc_segsum(x, src3d, dst4d)
    h1 = _tc_layer(x, s1, W1, b1d)
    s2 = _sc_segsum(h1, src3d, dst4d)
    h2, nb, y = _tc_stage2(h1, s2, x, batch2d, prototypes, W2, b2d,
                           Wm1, bm1d, Wm2, bm2d)
    nbrep = jnp.broadcast_to(nb, (_N, _D))
    eb = _sc_edge_bern(nbrep, src.reshape(_NW, _EPT // _K, 1, _K),
                       dst.reshape(_NW, _EPT // _K, 1, _K))
    eb = eb.reshape(_E, 16)[:, 0]
    s3 = _sc_segsum(y, src3d, dst4d)
    g1, z = _tc_stage3(x, s3, nb, W1, b1d)
    s4 = _sc_segsum(z, src3d, dst4d)
    se, sim, dsim, kl, nce = _tc_stage4(g1, s4, nb, eb.reshape(_E // _D, _D),
                                        batch2d, prototypes, W2, b2d)

    return (kl[0, 0], nce[0, 0], sim, nb, eb.reshape(_E, 1), dsim, se, h2)
